# baseline (speedup 1.0000x reference)
import functools

import jax
import jax.numpy as jnp
from jax import lax
from jax.experimental import pallas as pl
from jax.experimental.pallas import tpu as pltpu
from jax.experimental.pallas import tpu_sc as plsc

B = 16384
NBR = 50
D = 16
LANES = 128
NW = 32
BPW = B // NW
CB = 64
NCHUNK = BPW // CB
IDX_ROWS = CB * NBR // LANES
UROWS = BPW // LANES


def _sc_gather_one(idx_h, tab_h, out_h, idx_v, rows_v, sem):
    wid = lax.axis_index("s") * 2 + lax.axis_index("c")
    base = wid * BPW
    pltpu.sync_copy(idx_h.at[pl.ds(base, BPW)], idx_v)
    cps = [pltpu.async_copy(tab_h.at[idx_v.at[pl.ds(k * LANES, LANES)]],
                            rows_v.at[pl.ds(k * LANES, LANES)], sem)
           for k in range(UROWS)]
    for cp in cps:
        cp.wait()
    pltpu.sync_copy(rows_v, out_h.at[pl.ds(base, BPW)])


def _sc_body(user_h, nbr_h, uembs_h,
             uout_h, nsum_h,
             uidx_v, urows_v, nidx_v, nrows_v, nsum_v, semu, sem0, sem1):
    wid = lax.axis_index("s") * 2 + lax.axis_index("c")
    base = wid * BPW
    sems = (sem0, sem1)

    def issue(c, p):
        pltpu.sync_copy(
            nbr_h.at[pl.ds(base * NBR + c * CB * NBR, CB * NBR)],
            nidx_v.at[p])
        return [pltpu.async_copy(
                    uembs_h.at[nidx_v.at[p, pl.ds(k * LANES, LANES)]],
                    nrows_v.at[p, pl.ds(k * LANES, LANES)], sems[p])
                for k in range(IDX_ROWS)]

    pending = issue(0, 0)
    _sc_gather_one(user_h, uembs_h, uout_h, uidx_v, urows_v, semu)

    for c in range(NCHUNK):
        p = c % 2
        nxt = issue(c + 1, 1 - p) if c + 1 < NCHUNK else None
        for cp in pending:
            cp.wait()
        pending = nxt

        def rbody(b, rc):
            o = b * NBR
            accs = [nrows_v[p, o + j, :] for j in range(4)]
            for j in range(4, NBR):
                accs[j % 4] = accs[j % 4] + nrows_v[p, o + j, :]
            nsum_v[b, :] = (accs[0] + accs[1]) + (accs[2] + accs[3])
            return rc

        lax.fori_loop(0, CB, rbody, 0)
        pltpu.sync_copy(nsum_v, nsum_h.at[pl.ds(base + c * CB, CB)])


_sc_call = functools.partial(
    pl.kernel,
    mesh=plsc.VectorSubcoreMesh(core_axis_name="c", subcore_axis_name="s"),
    compiler_params=pltpu.CompilerParams(use_tc_tiling_on_sc=False),
    out_type=(
        jax.ShapeDtypeStruct((B, D), jnp.float32),
        jax.ShapeDtypeStruct((B, D), jnp.float32),
    ),
    scratch_types=(
        pltpu.VMEM((BPW,), jnp.int32),
        pltpu.VMEM((BPW, D), jnp.float32),
        pltpu.VMEM((2, CB * NBR), jnp.int32),
        pltpu.VMEM((2, CB * NBR, D), jnp.float32),
        pltpu.VMEM((CB, D), jnp.float32),
        pltpu.SemaphoreType.DMA,
        pltpu.SemaphoreType.DMA,
        pltpu.SemaphoreType.DMA,
    ),
)(_sc_body)


def _sc_item_body(item_h, iembs_h, iout_h, idx_v, rows_v, sem):
    _sc_gather_one(item_h, iembs_h, iout_h, idx_v, rows_v, sem)


_sc_item_call = functools.partial(
    pl.kernel,
    mesh=plsc.VectorSubcoreMesh(core_axis_name="c", subcore_axis_name="s"),
    compiler_params=pltpu.CompilerParams(use_tc_tiling_on_sc=False),
    out_type=jax.ShapeDtypeStruct((B, D), jnp.float32),
    scratch_types=(
        pltpu.VMEM((BPW,), jnp.int32),
        pltpu.VMEM((BPW, D), jnp.float32),
        pltpu.SemaphoreType.DMA,
    ),
)(_sc_item_body)


RPL = 8192
RPG = (10 ** 6 + RPL - 1) // RPL
TAB_ROWS = RPG * RPL


def _repack_body(xt_r, out_r):
    x3 = xt_r[...].reshape(D, 8, RPL // 8)
    m = x3.swapaxes(0, 1).reshape(8 * D, RPL // 8)
    out_r[...] = m.T


def _repack(tab_t):
    return pl.pallas_call(
        _repack_body,
        grid=(RPG,),
        in_specs=[pl.BlockSpec((D, RPL), lambda g: (0, g))],
        out_specs=pl.BlockSpec((RPL // 8, 8 * D), lambda g: (g, 0)),
        out_shape=jax.ShapeDtypeStruct((RPG * RPL // 8, 8 * D), jnp.float32),
    )(tab_t)


def _remap(j):
    return (j & ~8191) | ((j & 1023) << 3) | ((j >> 10) & 7)


TB = 2048


def _tc_body(uemb_r, iemb_r, nsum_r, nbr_r, logit_r, uet_r, nbet_r, iet_r):
    ue = uemb_r[...]
    ie = iemb_r[...]
    logit_r[...] = jnp.sum(ue * ie, axis=-1)
    cnt = jnp.sum((nbr_r[...] == 0).astype(jnp.float32), axis=-1)
    ln = jnp.float32(NBR) - cnt
    nbe = nsum_r[...] / ln[:, None]
    uet_r[...] = ue.T
    nbet_r[...] = nbe.T
    iet_r[...] = ie.T


def _tc_call(uemb, iemb, nsum, nbr):
    return pl.pallas_call(
        _tc_body,
        grid=(B // TB,),
        in_specs=[
            pl.BlockSpec((TB, D), lambda i: (i, 0)),
            pl.BlockSpec((TB, D), lambda i: (i, 0)),
            pl.BlockSpec((TB, D), lambda i: (i, 0)),
            pl.BlockSpec((TB, NBR), lambda i: (i, 0)),
        ],
        out_specs=[
            pl.BlockSpec((TB,), lambda i: (i,)),
            pl.BlockSpec((D, TB), lambda i: (0, i)),
            pl.BlockSpec((D, TB), lambda i: (0, i)),
            pl.BlockSpec((D, TB), lambda i: (0, i)),
        ],
        out_shape=[
            jax.ShapeDtypeStruct((B,), jnp.float32),
            jax.ShapeDtypeStruct((D, B), jnp.float32),
            jax.ShapeDtypeStruct((D, B), jnp.float32),
            jax.ShapeDtypeStruct((D, B), jnp.float32),
        ],
    )(uemb, iemb, nsum, nbr)


def kernel(user, u_ir, nbr, item, rate, user_embs, item_embs):
    utab = _repack(user_embs.T).reshape(TAB_ROWS, D)
    nbr_flat = _remap(nbr).reshape(B * NBR)
    uemb, nsum = _sc_call(_remap(user), nbr_flat, utab)
    itab = _repack(item_embs.T).reshape(TAB_ROWS, D)
    iemb = _sc_item_call(_remap(item), itab)
    logits, uet, nbet, iet = _tc_call(uemb, iemb, nsum, nbr)
    return (logits, uet.T, nbet.T, iet.T)


# speedup vs baseline: 1.2511x; 1.2511x over previous
import functools

import jax
import jax.numpy as jnp
from jax import lax
from jax.experimental import pallas as pl
from jax.experimental.pallas import tpu as pltpu
from jax.experimental.pallas import tpu_sc as plsc

B = 16384
NBR = 50
D = 16
LANES = 128
NW = 32
BPW = B // NW
CB = 64
NCHUNK = BPW // CB
IDX_ROWS = CB * NBR // LANES
UROWS = BPW // LANES


def _sc_gather_one(idx_h, tab_h, out_h, idx_v, rows_v, sem):
    wid = lax.axis_index("s") * 2 + lax.axis_index("c")
    base = wid * BPW
    pltpu.sync_copy(idx_h.at[pl.ds(base, BPW)], idx_v)
    cps = [pltpu.async_copy(tab_h.at[idx_v.at[pl.ds(k * LANES, LANES)]],
                            rows_v.at[pl.ds(k * LANES, LANES)], sem)
           for k in range(UROWS)]
    for cp in cps:
        cp.wait()
    pltpu.sync_copy(rows_v, out_h.at[pl.ds(base, BPW)])


def _sc_body(user_h, nbr_h, uembs_h,
             uout_h, nsum_h,
             uidx_v, urows_v, nidx_v, nrows_v, nsum_v, semu, sem0, sem1):
    wid = lax.axis_index("s") * 2 + lax.axis_index("c")
    base = wid * BPW
    sems = (sem0, sem1)

    def issue(c, p):
        pltpu.sync_copy(
            nbr_h.at[pl.ds(base * NBR + c * CB * NBR, CB * NBR)],
            nidx_v.at[p])
        return [pltpu.async_copy(
                    uembs_h.at[nidx_v.at[p, pl.ds(k * LANES, LANES)]],
                    nrows_v.at[p, pl.ds(k * LANES, LANES)], sems[p])
                for k in range(IDX_ROWS)]

    pending = issue(0, 0)
    _sc_gather_one(user_h, uembs_h, uout_h, uidx_v, urows_v, semu)

    for c in range(NCHUNK):
        p = c % 2
        nxt = issue(c + 1, 1 - p) if c + 1 < NCHUNK else None
        for cp in pending:
            cp.wait()
        pending = nxt

        def rbody(b, rc):
            o = b * NBR
            accs = [nrows_v[p, o + j, :] for j in range(4)]
            for j in range(4, NBR):
                accs[j % 4] = accs[j % 4] + nrows_v[p, o + j, :]
            nsum_v[b, :] = (accs[0] + accs[1]) + (accs[2] + accs[3])
            return rc

        lax.fori_loop(0, CB, rbody, 0)
        pltpu.sync_copy(nsum_v, nsum_h.at[pl.ds(base + c * CB, CB)])


_sc_call = functools.partial(
    pl.kernel,
    mesh=plsc.VectorSubcoreMesh(core_axis_name="c", subcore_axis_name="s"),
    compiler_params=pltpu.CompilerParams(use_tc_tiling_on_sc=False),
    out_type=(
        jax.ShapeDtypeStruct((B, D), jnp.float32),
        jax.ShapeDtypeStruct((B, D), jnp.float32),
    ),
    scratch_types=(
        pltpu.VMEM((BPW,), jnp.int32),
        pltpu.VMEM((BPW, D), jnp.float32),
        pltpu.VMEM((2, CB * NBR), jnp.int32),
        pltpu.VMEM((2, CB * NBR, D), jnp.float32),
        pltpu.VMEM((CB, D), jnp.float32),
        pltpu.SemaphoreType.DMA,
        pltpu.SemaphoreType.DMA,
        pltpu.SemaphoreType.DMA,
    ),
)(_sc_body)


def _sc_item_body(item_h, iembs_h, iout_h, idx_v, rows_v, sem):
    _sc_gather_one(item_h, iembs_h, iout_h, idx_v, rows_v, sem)


_sc_item_call = functools.partial(
    pl.kernel,
    mesh=plsc.VectorSubcoreMesh(core_axis_name="c", subcore_axis_name="s"),
    compiler_params=pltpu.CompilerParams(use_tc_tiling_on_sc=False),
    out_type=jax.ShapeDtypeStruct((B, D), jnp.float32),
    scratch_types=(
        pltpu.VMEM((BPW,), jnp.int32),
        pltpu.VMEM((BPW, D), jnp.float32),
        pltpu.SemaphoreType.DMA,
    ),
)(_sc_item_body)


RPL = 16384
RPG = (10 ** 6 + RPL - 1) // RPL
TAB_ROWS = RPG * RPL


def _repack_body(xt_r, out_r):
    x3 = xt_r[...].reshape(D, 8, RPL // 8)
    m = x3.swapaxes(0, 1).reshape(8 * D, RPL // 8)
    out_r[...] = m.T


def _repack(tab_t):
    return pl.pallas_call(
        _repack_body,
        grid=(RPG,),
        in_specs=[pl.BlockSpec((D, RPL), lambda g: (0, g))],
        out_specs=pl.BlockSpec((RPL // 8, 8 * D), lambda g: (g, 0)),
        out_shape=jax.ShapeDtypeStruct((RPG * RPL // 8, 8 * D), jnp.float32),
    )(tab_t)


_RCH = RPL // 8
_RSH = _RCH.bit_length() - 1


def _remap(j):
    return (j & ~(RPL - 1)) | ((j & (_RCH - 1)) << 3) | ((j >> _RSH) & 7)


TB = 2048


def _tc_body(uemb_r, iemb_r, nsum_r, nbr_r, logit_r, uet_r, nbet_r, iet_r):
    ue = uemb_r[...]
    ie = iemb_r[...]
    logit_r[...] = jnp.sum(ue * ie, axis=-1)
    cnt = jnp.sum((nbr_r[...] == 0).astype(jnp.float32), axis=-1)
    ln = jnp.float32(NBR) - cnt
    nbe = nsum_r[...] / ln[:, None]
    uet_r[...] = ue.T
    nbet_r[...] = nbe.T
    iet_r[...] = ie.T


def _tc_call(uemb, iemb, nsum, nbr):
    return pl.pallas_call(
        _tc_body,
        grid=(B // TB,),
        in_specs=[
            pl.BlockSpec((TB, D), lambda i: (i, 0)),
            pl.BlockSpec((TB, D), lambda i: (i, 0)),
            pl.BlockSpec((TB, D), lambda i: (i, 0)),
            pl.BlockSpec((TB, NBR), lambda i: (i, 0)),
        ],
        out_specs=[
            pl.BlockSpec((TB,), lambda i: (i,)),
            pl.BlockSpec((D, TB), lambda i: (0, i)),
            pl.BlockSpec((D, TB), lambda i: (0, i)),
            pl.BlockSpec((D, TB), lambda i: (0, i)),
        ],
        out_shape=[
            jax.ShapeDtypeStruct((B,), jnp.float32),
            jax.ShapeDtypeStruct((D, B), jnp.float32),
            jax.ShapeDtypeStruct((D, B), jnp.float32),
            jax.ShapeDtypeStruct((D, B), jnp.float32),
        ],
    )(uemb, iemb, nsum, nbr)


def kernel(user, u_ir, nbr, item, rate, user_embs, item_embs):
    utab = _repack(user_embs.T).reshape(TAB_ROWS, D)
    nbr_flat = _remap(nbr).reshape(B * NBR)
    uemb, nsum = _sc_call(_remap(user), nbr_flat, utab)
    itab = _repack(item_embs.T).reshape(TAB_ROWS, D)
    iemb = _sc_item_call(_remap(item), itab)
    logits, uet, nbet, iet = _tc_call(uemb, iemb, nsum, nbr)
    return (logits, uet.T, nbet.T, iet.T)


# speedup vs baseline: 1.4458x; 1.1556x over previous
import functools

import jax
import jax.numpy as jnp
from jax import lax
from jax.experimental import pallas as pl
from jax.experimental.pallas import tpu as pltpu
from jax.experimental.pallas import tpu_sc as plsc

B = 16384
NBR = 50
D = 16
LANES = 128
NW = 32
BPW = B // NW
CB = 64
NCHUNK = BPW // CB
IDX_ROWS = CB * NBR // LANES
UROWS = BPW // LANES


def _sc_gather_one(idx_h, tab_h, out_h, idx_v, rows_v, sem):
    wid = lax.axis_index("s") * 2 + lax.axis_index("c")
    base = wid * BPW
    pltpu.sync_copy(idx_h.at[pl.ds(base, BPW)], idx_v)
    cps = [pltpu.async_copy(tab_h.at[idx_v.at[pl.ds(k * LANES, LANES)]],
                            rows_v.at[pl.ds(k * LANES, LANES)], sem)
           for k in range(UROWS)]
    for cp in cps:
        cp.wait()
    pltpu.sync_copy(rows_v, out_h.at[pl.ds(base, BPW)])


def _sc_body(user_h, nbr_h, uembs_h,
             uout_h, nsum_h,
             uidx_v, urows_v, nidx_v, nrows_v, nsum_v, semu, sem0, sem1):
    wid = lax.axis_index("s") * 2 + lax.axis_index("c")
    base = wid * BPW
    sems = (sem0, sem1)

    def issue(c, p):
        pltpu.sync_copy(
            nbr_h.at[pl.ds(base * NBR + c * CB * NBR, CB * NBR)],
            nidx_v.at[p])
        return [pltpu.async_copy(
                    uembs_h.at[nidx_v.at[p, pl.ds(k * LANES, LANES)]],
                    nrows_v.at[p, pl.ds(k * LANES, LANES)], sems[p])
                for k in range(IDX_ROWS)]

    pending = issue(0, 0)
    _sc_gather_one(user_h, uembs_h, uout_h, uidx_v, urows_v, semu)

    for c in range(NCHUNK):
        p = c % 2
        nxt = issue(c + 1, 1 - p) if c + 1 < NCHUNK else None
        for cp in pending:
            cp.wait()
        pending = nxt

        def rbody(b, rc):
            o = b * NBR
            accs = [nrows_v[p, o + j, :] for j in range(4)]
            for j in range(4, NBR):
                accs[j % 4] = accs[j % 4] + nrows_v[p, o + j, :]
            nsum_v[b, :] = (accs[0] + accs[1]) + (accs[2] + accs[3])
            return rc

        lax.fori_loop(0, CB, rbody, 0)
        pltpu.sync_copy(nsum_v, nsum_h.at[pl.ds(base + c * CB, CB)])


_sc_call = functools.partial(
    pl.kernel,
    mesh=plsc.VectorSubcoreMesh(core_axis_name="c", subcore_axis_name="s"),
    compiler_params=pltpu.CompilerParams(use_tc_tiling_on_sc=False),
    out_type=(
        jax.ShapeDtypeStruct((B, D), jnp.float32),
        jax.ShapeDtypeStruct((B, D), jnp.float32),
    ),
    scratch_types=(
        pltpu.VMEM((BPW,), jnp.int32),
        pltpu.VMEM((BPW, D), jnp.float32),
        pltpu.VMEM((2, CB * NBR), jnp.int32),
        pltpu.VMEM((2, CB * NBR, D), jnp.float32),
        pltpu.VMEM((CB, D), jnp.float32),
        pltpu.SemaphoreType.DMA,
        pltpu.SemaphoreType.DMA,
        pltpu.SemaphoreType.DMA,
    ),
)(_sc_body)


def _sc_item_body(item_h, iembs_h, iout_h, idx_v, rows_v, sem):
    _sc_gather_one(item_h, iembs_h, iout_h, idx_v, rows_v, sem)


_sc_item_call = functools.partial(
    pl.kernel,
    mesh=plsc.VectorSubcoreMesh(core_axis_name="c", subcore_axis_name="s"),
    compiler_params=pltpu.CompilerParams(use_tc_tiling_on_sc=False),
    out_type=jax.ShapeDtypeStruct((B, D), jnp.float32),
    scratch_types=(
        pltpu.VMEM((BPW,), jnp.int32),
        pltpu.VMEM((BPW, D), jnp.float32),
        pltpu.SemaphoreType.DMA,
    ),
)(_sc_item_body)


RPL = 32768
RPG = (10 ** 6 + RPL - 1) // RPL
TAB_ROWS = RPG * RPL


def _repack_body(xt_r, out_r):
    x3 = xt_r[...].reshape(D, 8, RPL // 8)
    m = x3.swapaxes(0, 1).reshape(8 * D, RPL // 8)
    out_r[...] = m.T


def _repack(tab_t):
    return pl.pallas_call(
        _repack_body,
        grid=(RPG,),
        in_specs=[pl.BlockSpec((D, RPL), lambda g: (0, g))],
        out_specs=pl.BlockSpec((RPL // 8, 8 * D), lambda g: (g, 0)),
        out_shape=jax.ShapeDtypeStruct((RPG * RPL // 8, 8 * D), jnp.float32),
    )(tab_t)


_RCH = RPL // 8
_RSH = _RCH.bit_length() - 1


def _remap(j):
    return (j & ~(RPL - 1)) | ((j & (_RCH - 1)) << 3) | ((j >> _RSH) & 7)


TB = 2048


def _tc_body(uemb_r, iemb_r, nsum_r, nbr_r, logit_r, uet_r, nbet_r, iet_r):
    ue = uemb_r[...]
    ie = iemb_r[...]
    logit_r[...] = jnp.sum(ue * ie, axis=-1)
    cnt = jnp.sum((nbr_r[...] == 0).astype(jnp.float32), axis=-1)
    ln = jnp.float32(NBR) - cnt
    nbe = nsum_r[...] / ln[:, None]
    uet_r[...] = ue.T
    nbet_r[...] = nbe.T
    iet_r[...] = ie.T


def _tc_call(uemb, iemb, nsum, nbr):
    return pl.pallas_call(
        _tc_body,
        grid=(B // TB,),
        in_specs=[
            pl.BlockSpec((TB, D), lambda i: (i, 0)),
            pl.BlockSpec((TB, D), lambda i: (i, 0)),
            pl.BlockSpec((TB, D), lambda i: (i, 0)),
            pl.BlockSpec((TB, NBR), lambda i: (i, 0)),
        ],
        out_specs=[
            pl.BlockSpec((TB,), lambda i: (i,)),
            pl.BlockSpec((D, TB), lambda i: (0, i)),
            pl.BlockSpec((D, TB), lambda i: (0, i)),
            pl.BlockSpec((D, TB), lambda i: (0, i)),
        ],
        out_shape=[
            jax.ShapeDtypeStruct((B,), jnp.float32),
            jax.ShapeDtypeStruct((D, B), jnp.float32),
            jax.ShapeDtypeStruct((D, B), jnp.float32),
            jax.ShapeDtypeStruct((D, B), jnp.float32),
        ],
    )(uemb, iemb, nsum, nbr)


def kernel(user, u_ir, nbr, item, rate, user_embs, item_embs):
    utab = _repack(user_embs.T).reshape(TAB_ROWS, D)
    nbr_flat = _remap(nbr).reshape(B * NBR)
    uemb, nsum = _sc_call(_remap(user), nbr_flat, utab)
    itab = _repack(item_embs.T).reshape(TAB_ROWS, D)
    iemb = _sc_item_call(_remap(item), itab)
    logits, uet, nbet, iet = _tc_call(uemb, iemb, nsum, nbr)
    return (logits, uet.T, nbet.T, iet.T)


# speedup vs baseline: 1.4812x; 1.0245x over previous
import functools

import jax
import jax.numpy as jnp
from jax import lax
from jax.experimental import pallas as pl
from jax.experimental.pallas import tpu as pltpu
from jax.experimental.pallas import tpu_sc as plsc

B = 16384
NBR = 50
D = 16
LANES = 128
NW = 32
BPW = B // NW
CB = 64
NCHUNK = BPW // CB
IDX_ROWS = CB * NBR // LANES
UROWS = BPW // LANES


def _sc_gather_one(idx_h, tab_h, out_h, idx_v, rows_v, sem):
    wid = lax.axis_index("s") * 2 + lax.axis_index("c")
    base = wid * BPW
    pltpu.sync_copy(idx_h.at[pl.ds(base, BPW)], idx_v)
    cps = [pltpu.async_copy(tab_h.at[idx_v.at[pl.ds(k * LANES, LANES)]],
                            rows_v.at[pl.ds(k * LANES, LANES)], sem)
           for k in range(UROWS)]
    for cp in cps:
        cp.wait()
    pltpu.sync_copy(rows_v, out_h.at[pl.ds(base, BPW)])


def _sc_body(user_h, nbr_h, uembs_h,
             uout_h, nsum_h,
             uidx_v, urows_v, nidx_v, nrows_v, nsum_v, semu, sem0, sem1):
    wid = lax.axis_index("s") * 2 + lax.axis_index("c")
    base = wid * BPW
    sems = (sem0, sem1)

    def issue(c, p):
        pltpu.sync_copy(
            nbr_h.at[pl.ds(base * NBR + c * CB * NBR, CB * NBR)],
            nidx_v.at[p])
        return [pltpu.async_copy(
                    uembs_h.at[nidx_v.at[p, pl.ds(k * LANES, LANES)]],
                    nrows_v.at[p, pl.ds(k * LANES, LANES)], sems[p])
                for k in range(IDX_ROWS)]

    pending = issue(0, 0)
    _sc_gather_one(user_h, uembs_h, uout_h, uidx_v, urows_v, semu)

    for c in range(NCHUNK):
        p = c % 2
        nxt = issue(c + 1, 1 - p) if c + 1 < NCHUNK else None
        for cp in pending:
            cp.wait()
        pending = nxt

        def rbody(b, rc):
            o = b * NBR
            accs = [nrows_v[p, o + j, :] for j in range(4)]
            for j in range(4, NBR):
                accs[j % 4] = accs[j % 4] + nrows_v[p, o + j, :]
            nsum_v[b, :] = (accs[0] + accs[1]) + (accs[2] + accs[3])
            return rc

        lax.fori_loop(0, CB, rbody, 0)
        pltpu.sync_copy(nsum_v, nsum_h.at[pl.ds(base + c * CB, CB)])


_sc_call = functools.partial(
    pl.kernel,
    mesh=plsc.VectorSubcoreMesh(core_axis_name="c", subcore_axis_name="s"),
    compiler_params=pltpu.CompilerParams(use_tc_tiling_on_sc=False),
    out_type=(
        jax.ShapeDtypeStruct((B, D), jnp.float32),
        jax.ShapeDtypeStruct((B, D), jnp.float32),
    ),
    scratch_types=(
        pltpu.VMEM((BPW,), jnp.int32),
        pltpu.VMEM((BPW, D), jnp.float32),
        pltpu.VMEM((2, CB * NBR), jnp.int32),
        pltpu.VMEM((2, CB * NBR, D), jnp.float32),
        pltpu.VMEM((CB, D), jnp.float32),
        pltpu.SemaphoreType.DMA,
        pltpu.SemaphoreType.DMA,
        pltpu.SemaphoreType.DMA,
    ),
)(_sc_body)


def _sc_item_body(item_h, iembs_h, iout_h, idx_v, rows_v, sem):
    _sc_gather_one(item_h, iembs_h, iout_h, idx_v, rows_v, sem)


_sc_item_call = functools.partial(
    pl.kernel,
    mesh=plsc.VectorSubcoreMesh(core_axis_name="c", subcore_axis_name="s"),
    compiler_params=pltpu.CompilerParams(use_tc_tiling_on_sc=False),
    out_type=jax.ShapeDtypeStruct((B, D), jnp.float32),
    scratch_types=(
        pltpu.VMEM((BPW,), jnp.int32),
        pltpu.VMEM((BPW, D), jnp.float32),
        pltpu.SemaphoreType.DMA,
    ),
)(_sc_item_body)


RPL = 65536
RPG = (10 ** 6 + RPL - 1) // RPL
TAB_ROWS = RPG * RPL


def _repack_body(xt_r, out_r):
    x3 = xt_r[...].reshape(D, 8, RPL // 8)
    m = x3.swapaxes(0, 1).reshape(8 * D, RPL // 8)
    out_r[...] = m.T


def _repack(tab_t):
    return pl.pallas_call(
        _repack_body,
        grid=(RPG,),
        in_specs=[pl.BlockSpec((D, RPL), lambda g: (0, g))],
        out_specs=pl.BlockSpec((RPL // 8, 8 * D), lambda g: (g, 0)),
        out_shape=jax.ShapeDtypeStruct((RPG * RPL // 8, 8 * D), jnp.float32),
    )(tab_t)


_RCH = RPL // 8
_RSH = _RCH.bit_length() - 1


def _remap(j):
    return (j & ~(RPL - 1)) | ((j & (_RCH - 1)) << 3) | ((j >> _RSH) & 7)


TB = 2048


def _tc_body(uemb_r, iemb_r, nsum_r, nbr_r, logit_r, uet_r, nbet_r, iet_r):
    ue = uemb_r[...]
    ie = iemb_r[...]
    logit_r[...] = jnp.sum(ue * ie, axis=-1)
    cnt = jnp.sum((nbr_r[...] == 0).astype(jnp.float32), axis=-1)
    ln = jnp.float32(NBR) - cnt
    nbe = nsum_r[...] / ln[:, None]
    uet_r[...] = ue.T
    nbet_r[...] = nbe.T
    iet_r[...] = ie.T


def _tc_call(uemb, iemb, nsum, nbr):
    return pl.pallas_call(
        _tc_body,
        grid=(B // TB,),
        in_specs=[
            pl.BlockSpec((TB, D), lambda i: (i, 0)),
            pl.BlockSpec((TB, D), lambda i: (i, 0)),
            pl.BlockSpec((TB, D), lambda i: (i, 0)),
            pl.BlockSpec((TB, NBR), lambda i: (i, 0)),
        ],
        out_specs=[
            pl.BlockSpec((TB,), lambda i: (i,)),
            pl.BlockSpec((D, TB), lambda i: (0, i)),
            pl.BlockSpec((D, TB), lambda i: (0, i)),
            pl.BlockSpec((D, TB), lambda i: (0, i)),
        ],
        out_shape=[
            jax.ShapeDtypeStruct((B,), jnp.float32),
            jax.ShapeDtypeStruct((D, B), jnp.float32),
            jax.ShapeDtypeStruct((D, B), jnp.float32),
            jax.ShapeDtypeStruct((D, B), jnp.float32),
        ],
    )(uemb, iemb, nsum, nbr)


def kernel(user, u_ir, nbr, item, rate, user_embs, item_embs):
    utab = _repack(user_embs.T).reshape(TAB_ROWS, D)
    nbr_flat = _remap(nbr).reshape(B * NBR)
    uemb, nsum = _sc_call(_remap(user), nbr_flat, utab)
    itab = _repack(item_embs.T).reshape(TAB_ROWS, D)
    iemb = _sc_item_call(_remap(item), itab)
    logits, uet, nbet, iet = _tc_call(uemb, iemb, nsum, nbr)
    return (logits, uet.T, nbet.T, iet.T)
